# baseline (speedup 1.0000x reference)
import functools

import jax
import jax.numpy as jnp
from jax import lax
from jax.experimental import pallas as pl
from jax.experimental.pallas import tpu as pltpu
from jax.experimental.pallas import tpu_sc as plsc

NC = 2
NS = 16
NW = NC * NS
B = 128


def _make_sc_deg(N, C, n_pad):
    stripe = n_pad // NS

    mesh = plsc.VectorSubcoreMesh(core_axis_name="c", subcore_axis_name="s")

    @functools.partial(
        pl.kernel,
        mesh=mesh,
        out_type=jax.ShapeDtypeStruct((NC, n_pad), jnp.float32),
        scratch_types=[
            pltpu.VMEM((C, B), jnp.int32),
            pltpu.VMEM((B,), jnp.float32),
            pltpu.VMEM((stripe,), jnp.float32),
            pltpu.VMEM_SHARED((n_pad,), jnp.float32),
        ],
    )
    def deg_kernel(row_hbm, degp_hbm, idx_v, ones_v, zero_v, acc_sh):
        c = lax.axis_index("c")
        s = lax.axis_index("s")
        w = c * NS + s

        @pl.loop(0, stripe // 16)
        def _(i):
            zero_v[pl.ds(i * 16, 16)] = jnp.zeros((16,), jnp.float32)

        @pl.loop(0, B // 16)
        def _(j):
            ones_v[pl.ds(j * 16, 16)] = jnp.ones((16,), jnp.float32)

        pltpu.sync_copy(zero_v, acc_sh.at[pl.ds(s * stripe, stripe)])
        plsc.subcore_barrier()

        pltpu.sync_copy(row_hbm.at[w], idx_v)

        @pl.loop(0, C)
        def _(j):
            pltpu.sync_copy(ones_v, acc_sh.at[idx_v.at[j]], add=True)

        plsc.subcore_barrier()
        pltpu.sync_copy(
            acc_sh.at[pl.ds(s * stripe, stripe)],
            degp_hbm.at[c, pl.ds(s * stripe, stripe)],
        )

    return deg_kernel


def _make_sc_agg(N, D, C, half, rows_pad):
    stripe = rows_pad // NS
    zr = 80
    NBUF = 2

    mesh = plsc.VectorSubcoreMesh(core_axis_name="c", subcore_axis_name="s")

    @functools.partial(
        pl.kernel,
        mesh=mesh,
        out_type=jax.ShapeDtypeStruct((NC, rows_pad, D), jnp.float32),
        scratch_types=[
            pltpu.VMEM((C, B), jnp.int32),
            pltpu.VMEM((C, B), jnp.int32),
            pltpu.VMEM((NBUF, B, D), jnp.float32),
            pltpu.VMEM((zr, D), jnp.float32),
            pltpu.VMEM_SHARED((rows_pad, D), jnp.float32),
        ] + [pltpu.SemaphoreType.DMA] * NBUF,
    )
    def agg_kernel(g_hbm, rowg_hbm, cols_hbm, accp_hbm,
                   rowi_v, coli_v, rows_v, zero_v, acc_sh, *gsems):
        c = lax.axis_index("c")
        s = lax.axis_index("s")

        @pl.loop(0, zr)
        def _(i):
            for k in range(D // 16):
                zero_v[i, pl.ds(k * 16, 16)] = jnp.zeros((16,), jnp.float32)

        base = s * stripe
        off = 0
        while off < stripe:
            n = min(zr, stripe - off)
            pltpu.sync_copy(zero_v.at[pl.ds(0, n)],
                            acc_sh.at[pl.ds(base + off, n)])
            off += n

        pltpu.sync_copy(rowg_hbm.at[s], rowi_v)
        pltpu.sync_copy(cols_hbm.at[s], coli_v)

        node0 = c * half

        @pl.loop(0, C)
        def _(j):
            for k in range(B // 16):
                v = coli_v[j, pl.ds(k * 16, 16)]
                lv = v - node0
                ok = (lv >= 0) & (lv < half)
                coli_v[j, pl.ds(k * 16, 16)] = jnp.where(ok, lv, half)

        plsc.subcore_barrier()

        for b in range(NBUF):
            pltpu.async_copy(g_hbm.at[rowi_v.at[b]], rows_v.at[b], gsems[b])

        @pl.loop(0, C // NBUF)
        def _(g):
            j0 = g * NBUF
            for b in range(NBUF):
                j = j0 + b
                pltpu.make_async_copy(
                    g_hbm.at[rowi_v.at[j]], rows_v.at[b], gsems[b]).wait()
                pltpu.sync_copy(rows_v.at[b], acc_sh.at[coli_v.at[j]],
                                add=True)
                jn = j + NBUF

                @pl.when(jn < C)
                def _():
                    pltpu.async_copy(
                        g_hbm.at[rowi_v.at[jn]], rows_v.at[b], gsems[b])

        plsc.subcore_barrier()
        pltpu.sync_copy(
            acc_sh.at[pl.ds(s * stripe, stripe)],
            accp_hbm.at[c, pl.ds(s * stripe, stripe)],
        )

    return agg_kernel


def _tc_pre_body(x_ref, w_ref, b_ref, dp_ref, g_ref, dis_ref):
    h = lax.dot_general(
        x_ref[...], w_ref[...],
        (((1,), (1,)), ((), ())),
        preferred_element_type=jnp.float32,
    ) + b_ref[...]
    deg = dp_ref[0, :] + dp_ref[1, :]
    dis = lax.rsqrt(deg)
    g_ref[...] = dis[:, None] * h
    dis_ref[...] = dis[None, :]


def _tc_post_body(acc_ref, dis_ref, out_ref):
    out_ref[...] = dis_ref[0, :][:, None] * acc_ref[...]


def kernel(x, edge_index, W, b):
    N, D = x.shape
    E = edge_index.shape[1]
    row = edge_index[0]
    col = edge_index[1]

    Cd = -(-E // (NW * B))
    pad_d = NW * Cd * B - E
    C = -(-(-(-E // (NS * B))) // 4) * 4
    pad_a = NS * C * B - E
    n_pad = -(-N // (16 * NS)) * (16 * NS)
    half = -(-N // 2)
    rows_pad = -(-(half + 1) // (8 * NS)) * (8 * NS)

    rowd = jnp.concatenate(
        [row, jnp.full((pad_d,), N, jnp.int32)]).reshape(NW, Cd, B)
    rowg = jnp.concatenate(
        [row, jnp.zeros((pad_a,), jnp.int32)]).reshape(NS, C, B)
    cols = jnp.concatenate(
        [col, jnp.full((pad_a,), N, jnp.int32)]).reshape(NS, C, B)

    degp = _make_sc_deg(N, Cd, n_pad)(rowd)
    dp = degp[:, :N]

    BN = 512
    grid = (-(-N // BN),)
    g, dis = pl.pallas_call(
        _tc_pre_body,
        grid=grid,
        in_specs=[
            pl.BlockSpec((BN, D), lambda i: (i, 0)),
            pl.BlockSpec((D, D), lambda i: (0, 0)),
            pl.BlockSpec((1, D), lambda i: (0, 0)),
            pl.BlockSpec((2, BN), lambda i: (0, i)),
        ],
        out_specs=[
            pl.BlockSpec((BN, D), lambda i: (i, 0)),
            pl.BlockSpec((1, BN), lambda i: (0, i)),
        ],
        out_shape=[
            jax.ShapeDtypeStruct((N, D), jnp.float32),
            jax.ShapeDtypeStruct((1, N), jnp.float32),
        ],
    )(x, W, b[None, :], dp)

    accp = _make_sc_agg(N, D, C, half, rows_pad)(g, rowg, cols)
    acc = jnp.concatenate([accp[0, :half], accp[1, :N - half]], axis=0)

    out = pl.pallas_call(
        _tc_post_body,
        grid=grid,
        in_specs=[
            pl.BlockSpec((BN, D), lambda i: (i, 0)),
            pl.BlockSpec((1, BN), lambda i: (0, i)),
        ],
        out_specs=pl.BlockSpec((BN, D), lambda i: (i, 0)),
        out_shape=jax.ShapeDtypeStruct((N, D), jnp.float32),
    )(acc, dis)
    return out


# speedup vs baseline: 1.4617x; 1.4617x over previous
import functools

import jax
import jax.numpy as jnp
from jax import lax
from jax.experimental import pallas as pl
from jax.experimental.pallas import tpu as pltpu
from jax.experimental.pallas import tpu_sc as plsc

NC = 2
NS = 16
NW = NC * NS
B = 128
SUP = B


def _prefix16(ki, iota):
    x = ki
    for p in range(4):
        sh = 1 << p
        x = x + jnp.where(iota >= sh, x[jnp.maximum(iota - sh, 0)], 0)
    return x


def _lower_bound16(x, t):
    pos = jnp.zeros((16,), jnp.int32)
    for step in (8, 4, 2, 1):
        xv = x[pos + (step - 1)]
        pos = jnp.where(xv < t, pos + step, pos)
    return pos


def _make_sc_deg_part(N, Cd, n_pad, half, cap):
    stripe = n_pad // NS

    mesh = plsc.VectorSubcoreMesh(core_axis_name="c", subcore_axis_name="s")

    @functools.partial(
        pl.kernel,
        mesh=mesh,
        out_type=[
            jax.ShapeDtypeStruct((NC, n_pad), jnp.float32),
            jax.ShapeDtypeStruct((NC, NW, cap), jnp.int32),
            jax.ShapeDtypeStruct((NC, NW, cap), jnp.int32),
            jax.ShapeDtypeStruct((NC, NW, 16), jnp.int32),
        ],
        scratch_types=[
            pltpu.VMEM((Cd, B), jnp.int32),
            pltpu.VMEM((Cd, B), jnp.int32),
            pltpu.VMEM((Cd, B), jnp.int32),
            pltpu.VMEM((cap,), jnp.int32),
            pltpu.VMEM((cap,), jnp.int32),
            pltpu.VMEM((cap,), jnp.int32),
            pltpu.VMEM((cap,), jnp.int32),
            pltpu.VMEM((16,), jnp.int32),
            pltpu.VMEM((B,), jnp.float32),
            pltpu.VMEM((stripe,), jnp.float32),
            pltpu.VMEM_SHARED((n_pad,), jnp.float32),
        ],
    )
    def deg_part_kernel(rowd_hbm, rowg_hbm, colp_hbm,
                        degp_hbm, pr_hbm, pc_hbm, cnt_hbm,
                        rowd_v, rowg_v, colp_v,
                        lo_r, lo_c, hi_r, hi_c, cnt_v,
                        ones_v, zero_v, acc_sh):
        c = lax.axis_index("c")
        s = lax.axis_index("s")
        w = c * NS + s

        @pl.loop(0, stripe // 16)
        def _(i):
            zero_v[pl.ds(i * 16, 16)] = jnp.zeros((16,), jnp.float32)

        @pl.loop(0, B // 16)
        def _(j):
            ones_v[pl.ds(j * 16, 16)] = jnp.ones((16,), jnp.float32)

        pltpu.sync_copy(zero_v, acc_sh.at[pl.ds(s * stripe, stripe)])
        plsc.subcore_barrier()

        pltpu.sync_copy(rowd_hbm.at[w], rowd_v)
        pltpu.sync_copy(rowg_hbm.at[w], rowg_v)
        pltpu.sync_copy(colp_hbm.at[w], colp_v)

        @pl.loop(0, Cd)
        def _(j):
            pltpu.sync_copy(ones_v, acc_sh.at[rowd_v.at[j]], add=True)

        iota = lax.iota(jnp.int32, 16)
        t16 = iota + 1

        def part_body(j, offs):
            lo_off, hi_off = offs
            for k in range(B // 16):
                r = rowg_v[j, pl.ds(k * 16, 16)]
                cv = colp_v[j, pl.ds(k * 16, 16)]
                in_lo = cv < half
                ki = jnp.where(in_lo, 1, 0).astype(jnp.int32)
                x = _prefix16(ki, iota)
                nlo = x[15]
                src_lo = jnp.minimum(_lower_bound16(x, t16), 15)
                xh = t16 - x
                src_hi = jnp.minimum(_lower_bound16(xh, t16), 15)
                lo_r[pl.ds(lo_off, 16)] = r[src_lo]
                lo_c[pl.ds(lo_off, 16)] = cv[src_lo]
                hi_r[pl.ds(hi_off, 16)] = r[src_hi]
                hi_c[pl.ds(hi_off, 16)] = cv[src_hi] - half
                lo_off = lo_off + nlo
                hi_off = hi_off + (16 - nlo)
            return (lo_off, hi_off)

        lo_off, hi_off = pl.loop(0, Cd, init_carry=(0, 0))(part_body)

        dz = jnp.zeros((16,), jnp.int32)
        dh = jnp.full((16,), half, jnp.int32)
        for side_r, side_c, off in ((lo_r, lo_c, lo_off),
                                    (hi_r, hi_c, hi_off)):
            for k in range(B // 16):
                side_r[pl.ds(off + k * 16, 16)] = dz
                side_c[pl.ds(off + k * 16, 16)] = dh

        pltpu.sync_copy(lo_r, pr_hbm.at[0, w])
        pltpu.sync_copy(lo_c, pc_hbm.at[0, w])
        pltpu.sync_copy(hi_r, pr_hbm.at[1, w])
        pltpu.sync_copy(hi_c, pc_hbm.at[1, w])
        n_ch_lo = (lo_off + SUP - 1) >> 7
        n_ch_hi = (hi_off + SUP - 1) >> 7
        cnt_v[pl.ds(0, 16)] = jnp.full((16,), n_ch_lo, jnp.int32)
        pltpu.sync_copy(cnt_v, cnt_hbm.at[0, w])
        cnt_v[pl.ds(0, 16)] = jnp.full((16,), n_ch_hi, jnp.int32)
        pltpu.sync_copy(cnt_v, cnt_hbm.at[1, w])

        plsc.subcore_barrier()
        pltpu.sync_copy(
            acc_sh.at[pl.ds(s * stripe, stripe)],
            degp_hbm.at[c, pl.ds(s * stripe, stripe)],
        )

    return deg_part_kernel


def _make_sc_agg(N, D, capC, rows_pad):
    stripe = rows_pad // NS
    zr = 80

    mesh = plsc.VectorSubcoreMesh(core_axis_name="c", subcore_axis_name="s")

    @functools.partial(
        pl.kernel,
        mesh=mesh,
        out_type=jax.ShapeDtypeStruct((NC, rows_pad, D), jnp.float32),
        scratch_types=[
            pltpu.VMEM((capC, B), jnp.int32),
            pltpu.VMEM((capC, B), jnp.int32),
            pltpu.VMEM((16,), jnp.int32),
            pltpu.VMEM((B, D), jnp.float32),
            pltpu.VMEM((zr, D), jnp.float32),
            pltpu.VMEM_SHARED((rows_pad, D), jnp.float32),
            pltpu.SemaphoreType.DMA,
        ],
    )
    def agg_kernel(g_hbm, pr_hbm, pc_hbm, cnt_hbm, accp_hbm,
                   rowi_v, coli_v, cnt_v, rows_v, zero_v, acc_sh, sem):
        c = lax.axis_index("c")
        s = lax.axis_index("s")

        @pl.loop(0, zr)
        def _(i):
            for k in range(D // 16):
                zero_v[i, pl.ds(k * 16, 16)] = jnp.zeros((16,), jnp.float32)

        base = s * stripe
        off = 0
        while off < stripe:
            n = min(zr, stripe - off)
            pltpu.sync_copy(zero_v.at[pl.ds(0, n)],
                            acc_sh.at[pl.ds(base + off, n)])
            off += n
        plsc.subcore_barrier()

        for t in range(2):
            w = 2 * s + t
            pltpu.sync_copy(pr_hbm.at[c, w], rowi_v)
            pltpu.sync_copy(pc_hbm.at[c, w], coli_v)
            pltpu.sync_copy(cnt_hbm.at[c, w], cnt_v)
            nchunks = cnt_v[...][0]

            @pl.loop(0, nchunks)
            def _(j):
                pltpu.async_copy(g_hbm.at[rowi_v.at[j]], rows_v, sem).wait()
                pltpu.sync_copy(rows_v, acc_sh.at[coli_v.at[j]], add=True)

        plsc.subcore_barrier()
        pltpu.sync_copy(
            acc_sh.at[pl.ds(s * stripe, stripe)],
            accp_hbm.at[c, pl.ds(s * stripe, stripe)],
        )

    return agg_kernel


def _tc_pre_body(x_ref, w_ref, b_ref, dp_ref, g_ref, dis_ref):
    h = lax.dot_general(
        x_ref[...], w_ref[...],
        (((1,), (1,)), ((), ())),
        preferred_element_type=jnp.float32,
    ) + b_ref[...]
    deg = dp_ref[0, :] + dp_ref[1, :]
    dis = lax.rsqrt(deg)
    g_ref[...] = dis[:, None] * h
    dis_ref[...] = dis[None, :]


def _tc_post_body(acc_ref, dis_ref, out_ref):
    out_ref[...] = dis_ref[0, :][:, None] * acc_ref[...]


def kernel(x, edge_index, W, b):
    N, D = x.shape
    E = edge_index.shape[1]
    row = edge_index[0]
    col = edge_index[1]

    Cd = -(-E // (NW * B))
    ev = Cd * B
    pad_d = NW * ev - E
    cap = ev + B
    capC = cap // B
    n_pad = -(-N // (16 * NS)) * (16 * NS)
    half = -(-N // 2)
    rows_pad = -(-(half + 1) // (8 * NS)) * (8 * NS)

    rowd = jnp.concatenate(
        [row, jnp.full((pad_d,), N, jnp.int32)]).reshape(NW, Cd, B)
    rowg = jnp.concatenate(
        [row, jnp.zeros((pad_d,), jnp.int32)]).reshape(NW, Cd, B)
    colp = jnp.concatenate(
        [col, jnp.full((pad_d,), N, jnp.int32)]).reshape(NW, Cd, B)

    degp, pr, pc, cnt = _make_sc_deg_part(N, Cd, n_pad, half, cap)(
        rowd, rowg, colp)
    dp = degp[:, :N]
    pr = pr.reshape(NC, NW, capC, B)
    pc = pc.reshape(NC, NW, capC, B)

    BN = 512
    grid = (-(-N // BN),)
    g, dis = pl.pallas_call(
        _tc_pre_body,
        grid=grid,
        in_specs=[
            pl.BlockSpec((BN, D), lambda i: (i, 0)),
            pl.BlockSpec((D, D), lambda i: (0, 0)),
            pl.BlockSpec((1, D), lambda i: (0, 0)),
            pl.BlockSpec((2, BN), lambda i: (0, i)),
        ],
        out_specs=[
            pl.BlockSpec((BN, D), lambda i: (i, 0)),
            pl.BlockSpec((1, BN), lambda i: (0, i)),
        ],
        out_shape=[
            jax.ShapeDtypeStruct((N, D), jnp.float32),
            jax.ShapeDtypeStruct((1, N), jnp.float32),
        ],
    )(x, W, b[None, :], dp)

    accp = _make_sc_agg(N, D, capC, rows_pad)(g, pr, pc, cnt)
    acc = jnp.concatenate([accp[0, :half], accp[1, :N - half]], axis=0)

    out = pl.pallas_call(
        _tc_post_body,
        grid=grid,
        in_specs=[
            pl.BlockSpec((BN, D), lambda i: (i, 0)),
            pl.BlockSpec((1, BN), lambda i: (0, i)),
        ],
        out_specs=pl.BlockSpec((BN, D), lambda i: (i, 0)),
        out_shape=jax.ShapeDtypeStruct((N, D), jnp.float32),
    )(acc, dis)
    return out
